# baseline (speedup 1.0000x reference)
import jax
import jax.numpy as jnp
from jax.experimental import pallas as pl
from jax.experimental.pallas import tpu as pltpu


def _add_kernel(x_ref, pos_ref, o_ref):
    o_ref[...] = x_ref[...] + pos_ref[...]


def kernel(x, pos_table):
    B, P, E = x.shape
    x2 = x.reshape(B * P, E)
    out = pl.pallas_call(
        _add_kernel,
        grid=(B,),
        in_specs=[
            pl.BlockSpec((P, E), lambda b: (b, 0)),
            pl.BlockSpec((P, E), lambda b: (0, 0)),
        ],
        out_specs=pl.BlockSpec((P, E), lambda b: (b, 0)),
        out_shape=jax.ShapeDtypeStruct((B * P, E), x.dtype),
        compiler_params=pltpu.CompilerParams(
            dimension_semantics=("parallel",),
        ),
    )(x2, pos_table)
    return out.reshape(B, P, E)


# speedup vs baseline: 1.0335x; 1.0335x over previous
import jax
import jax.numpy as jnp
from jax.experimental import pallas as pl
from jax.experimental.pallas import tpu as pltpu

_PERIODS_PER_BLOCK = 2


def _add_kernel(x_ref, pos_ref, o_ref):
    P = pos_ref.shape[0]
    for k in range(_PERIODS_PER_BLOCK):
        sl = pl.ds(k * P, P)
        o_ref[sl, :] = x_ref[sl, :] + pos_ref[...]


def kernel(x, pos_table):
    B, P, E = x.shape
    R = _PERIODS_PER_BLOCK * P
    n_blocks = (B * P) // R
    x2 = x.reshape(B * P, E)
    out = pl.pallas_call(
        _add_kernel,
        grid=(n_blocks,),
        in_specs=[
            pl.BlockSpec((R, E), lambda b: (b, 0)),
            pl.BlockSpec((P, E), lambda b: (0, 0)),
        ],
        out_specs=pl.BlockSpec((R, E), lambda b: (b, 0)),
        out_shape=jax.ShapeDtypeStruct((B * P, E), x.dtype),
        compiler_params=pltpu.CompilerParams(
            dimension_semantics=("arbitrary",),
        ),
    )(x2, pos_table)
    return out.reshape(B, P, E)


# speedup vs baseline: 1.0445x; 1.0107x over previous
import jax
import jax.numpy as jnp
from jax.experimental import pallas as pl
from jax.experimental.pallas import tpu as pltpu

_PERIODS_PER_BLOCK = 4


def _add_kernel(x_ref, pos_ref, o_ref):
    P = pos_ref.shape[0]
    for k in range(_PERIODS_PER_BLOCK):
        sl = pl.ds(k * P, P)
        o_ref[sl, :] = x_ref[sl, :] + pos_ref[...]


def kernel(x, pos_table):
    B, P, E = x.shape
    R = _PERIODS_PER_BLOCK * P
    n_blocks = (B * P) // R
    x2 = x.reshape(B * P, E)
    out = pl.pallas_call(
        _add_kernel,
        grid=(n_blocks,),
        in_specs=[
            pl.BlockSpec((R, E), lambda b: (b, 0)),
            pl.BlockSpec((P, E), lambda b: (0, 0)),
        ],
        out_specs=pl.BlockSpec((R, E), lambda b: (b, 0)),
        out_shape=jax.ShapeDtypeStruct((B * P, E), x.dtype),
        compiler_params=pltpu.CompilerParams(
            dimension_semantics=("arbitrary",),
        ),
    )(x2, pos_table)
    return out.reshape(B, P, E)
